# baseline (speedup 1.0000x reference)
import jax
import jax.numpy as jnp
from jax.experimental import pallas as pl

N_GRAPHS = 128


def kernel(x, edge_index, edge_attr, edge_index_bond, edge_index_batch,
           W_u, W_v, W_e, W_rel, b_rel, W_root, a, W_gout, b_gout, a_bias):
    E_loc = edge_attr.shape[0]
    src, dst = edge_index_bond[0], edge_index_bond[1]
    seg = edge_index_batch
    n_iter = a.shape[-1]

    edge_u = x @ W_u
    edge_v = x @ W_v
    edge_uv = edge_attr @ W_e
    ea = (edge_u[edge_index[0]] + edge_v[edge_index[1]] + edge_uv) / 3.0

    out = ea
    out_list = []
    gout_list = []
    for n in range(n_iter):
        agg = jax.ops.segment_sum(out[src], dst, num_segments=E_loc)
        out = ea + agg
        p = out @ W_rel
        q = jax.ops.segment_sum(p[src], dst, num_segments=E_loc)
        x_conv = q + b_rel + out @ W_root
        m = jax.ops.segment_max(x_conv, seg, num_segments=N_GRAPHS)
        x_exp = jnp.exp(x_conv - m[seg])
        denom = jax.ops.segment_sum(x_exp, seg, num_segments=N_GRAPHS)
        scores_e = x_exp / (denom[seg] + 1e-16)
        gx = jax.ops.segment_sum(out * scores_e, seg, num_segments=N_GRAPHS)
        out_list.append(out)
        gout_list.append(jnp.tanh(gx @ W_gout + b_gout))

    gout_all = jnp.stack(gout_list, axis=-1)
    out_all = jnp.stack(out_list, axis=-1)
    sc = jnp.sum(gout_all * a, axis=1, keepdims=True) + a_bias
    sc = jax.nn.softmax(sc, axis=-1)
    sc_edge = sc[seg]
    out_final = jnp.sum(out_all * sc_edge, axis=-1)
    x_new = x + jax.ops.segment_sum(out_final, edge_index[1], num_segments=x.shape[0])
    return x_new


# speedup vs baseline: 1.0555x; 1.0555x over previous
import functools

import jax
import jax.numpy as jnp
from jax import lax
from jax.experimental import pallas as pl
from jax.experimental.pallas import tpu as pltpu
from jax.experimental.pallas import tpu_sc as plsc

N_GRAPHS = 128
G = 128
D = 128


def _fetch_scalar(vec_ref, n16, b):
    del n16
    return vec_ref[b][0]


def _mp_body(C, NBH, R, table, ea, srcp, ldstp, offs, outn,
             offs_v, sidx, lraw, lfix, rows, acc, sem):
    c = lax.axis_index("c")
    s = lax.axis_index("s")
    pltpu.sync_copy(offs, offs_v)
    pltpu.sync_copy(ea.at[pl.ds(c * C + s * R, R)], acc.at[pl.ds(s * R, R)])
    plsc.subcore_barrier()

    def blk(i, carry):
        b = c + 2 * i
        lo = _fetch_scalar(offs_v, 4, b)
        hi = _fetch_scalar(offs_v, 4, b + 1)
        n = hi - lo
        K = lax.div(n + 15, jnp.int32(16))
        start = lo + s * K
        end = jnp.minimum(start + K, hi)
        abase = start - lax.rem(start, jnp.int32(8))
        nch = jnp.where(end > start,
                        lax.div(end - abase + (G - 1), jnp.int32(G)),
                        jnp.int32(0))

        def ch(j, _):
            w = pl.multiple_of(abase + j * G, 8)
            pltpu.sync_copy(srcp.at[pl.ds(w, G)], sidx)
            pltpu.sync_copy(ldstp.at[pl.ds(w, G)], lraw)
            pltpu.async_copy(table.at[sidx], rows, sem).wait()
            for k in range(G // 16):
                gi = w + 16 * k + lax.iota(jnp.int32, 16)
                lv = lraw[pl.ds(16 * k, 16)]
                ok = (gi >= start) & (gi < end)
                lfixv = jnp.where(ok, lv, jnp.int32(C))
                pltpu.sync_copy(rows.at[pl.ds(16 * k, 16)], acc.at[lfixv],
                                add=True)
            return 0

        lax.fori_loop(0, nch, ch, 0)
        plsc.subcore_barrier()
        pltpu.sync_copy(acc.at[pl.ds(s * R, R)], outn.at[pl.ds(b * C + s * R, R)])

        @pl.when(i < NBH - 1)
        def _():
            pltpu.sync_copy(ea.at[pl.ds((b + 2) * C + s * R, R)],
                            acc.at[pl.ds(s * R, R)])

        plsc.subcore_barrier()
        return carry

    lax.fori_loop(0, NBH, blk, 0)


def _make_mp_kernel(EL, EP, C):
    NB = EL // C
    NBH = NB // 2
    R = C // 16
    mesh = plsc.VectorSubcoreMesh(core_axis_name="c", subcore_axis_name="s")
    body = functools.partial(_mp_body, C, NBH, R)
    return pl.kernel(
        body,
        out_type=jax.ShapeDtypeStruct((EL, D), jnp.float32),
        scratch_types=[
            pltpu.VMEM((64, 16), jnp.int32),
            pltpu.VMEM((G,), jnp.int32),
            pltpu.VMEM((G,), jnp.int32),
            pltpu.VMEM((G,), jnp.int32),
            pltpu.VMEM((G, D), jnp.float32),
            pltpu.VMEM_SHARED((C + 8, D), jnp.float32),
            pltpu.SemaphoreType.DMA,
        ],
        mesh=mesh,
    )


def _q_body(EH, R1, p, srcp, ldstp, offs, qout,
            offs_v, sidx, lraw, lfix, prows, zbuf, qacc, sem):
    c = lax.axis_index("c")
    s = lax.axis_index("s")
    pltpu.sync_copy(offs, offs_v)

    def zf(i, _):
        zbuf[pl.ds(16 * i, 16)] = jnp.zeros((16,), jnp.float32)
        return 0

    lax.fori_loop(0, 125, zf, 0)
    for t in range(5):
        pltpu.sync_copy(zbuf, qacc.at[pl.ds(s * R1 + 2000 * t, 2000)])
    plsc.subcore_barrier()

    lo = _fetch_scalar(offs_v, 1, c)
    hi = _fetch_scalar(offs_v, 1, c + 1)
    n = hi - lo
    K = lax.div(n + 15, jnp.int32(16))
    start = lo + s * K
    end = jnp.minimum(start + K, hi)
    abase = start - lax.rem(start, jnp.int32(8))
    nch = jnp.where(end > start,
                    lax.div(end - abase + (G - 1), jnp.int32(G)),
                    jnp.int32(0))

    def ch(j, _):
        w = pl.multiple_of(abase + j * G, 8)
        pltpu.sync_copy(srcp.at[pl.ds(w, G)], sidx)
        pltpu.sync_copy(ldstp.at[pl.ds(w, G)], lraw)
        for k in range(G // 16):
            gi = w + 16 * k + lax.iota(jnp.int32, 16)
            lv = lraw[pl.ds(16 * k, 16)]
            ok = (gi >= start) & (gi < end)
            lfix[pl.ds(16 * k, 16)] = jnp.where(ok, lv, jnp.int32(EH))
        pltpu.async_copy(p.at[sidx], prows, sem).wait()
        pltpu.sync_copy(prows, qacc.at[lfix], add=True)
        return 0

    lax.fori_loop(0, nch, ch, 0)
    plsc.subcore_barrier()
    for t in range(5):
        pltpu.sync_copy(qacc.at[pl.ds(s * R1 + 2000 * t, 2000)], zbuf)
        pltpu.sync_copy(zbuf, qout.at[pl.ds(c * EH + s * R1 + 2000 * t, 2000)])


def _make_q_kernel(EL, EP, EH):
    R1 = EH // 16
    mesh = plsc.VectorSubcoreMesh(core_axis_name="c", subcore_axis_name="s")
    body = functools.partial(_q_body, EH, R1)
    return pl.kernel(
        body,
        out_type=jax.ShapeDtypeStruct((EL,), jnp.float32),
        scratch_types=[
            pltpu.VMEM((16, 16), jnp.int32),
            pltpu.VMEM((G,), jnp.int32),
            pltpu.VMEM((G,), jnp.int32),
            pltpu.VMEM((G,), jnp.int32),
            pltpu.VMEM((G,), jnp.float32),
            pltpu.VMEM((2000,), jnp.float32),
            pltpu.VMEM_SHARED((EH + 8,), jnp.float32),
            pltpu.SemaphoreType.DMA,
        ],
        mesh=mesh,
    )


def kernel(x, edge_index, edge_attr, edge_index_bond, edge_index_batch,
           W_u, W_v, W_e, W_rel, b_rel, W_root, a, W_gout, b_gout, a_bias):
    EL = edge_attr.shape[0]
    src, dst = edge_index_bond[0], edge_index_bond[1]
    seg = edge_index_batch
    n_iter = a.shape[-1]

    C = 6400
    NB = EL // C
    EH = EL // 2

    dst_s, src_s = lax.sort_key_val(dst.astype(jnp.int32), src.astype(jnp.int32))
    ldst = (dst_s % C).astype(jnp.int32)
    ldst1 = (dst_s % EH).astype(jnp.int32)
    src_pad = jnp.concatenate([src_s, jnp.zeros((G,), jnp.int32)])
    ldst_pad = jnp.concatenate([ldst, jnp.full((G,), C, jnp.int32)])
    ldst1_pad = jnp.concatenate([ldst1, jnp.full((G,), EH, jnp.int32)])
    offs = jnp.searchsorted(dst_s, jnp.arange(NB + 1, dtype=jnp.int32) * C)
    offs = jnp.concatenate([offs.astype(jnp.int32),
                            jnp.zeros((64 - NB - 1,), jnp.int32)])
    offs = jnp.tile(offs[:, None], (1, 16))
    qoffs = jnp.searchsorted(dst_s, jnp.arange(3, dtype=jnp.int32) * EH)
    qoffs = jnp.concatenate([qoffs.astype(jnp.int32), jnp.zeros((13,), jnp.int32)])
    qoffs = jnp.tile(qoffs[:, None], (1, 16))

    EP = EL + G
    mp = _make_mp_kernel(EL, EP, C)
    qk = _make_q_kernel(EL, EP, EH)

    edge_u = x @ W_u
    edge_v = x @ W_v
    edge_uv = edge_attr @ W_e
    ea = (edge_u[edge_index[0]] + edge_v[edge_index[1]] + edge_uv) / 3.0

    out = ea
    out_list = []
    gout_list = []
    for n in range(n_iter):
        out = mp(out, ea, src_pad, ldst_pad, offs)
        p = (out @ W_rel).reshape(EL)
        q = jax.ops.segment_sum(p[src], dst, num_segments=EL).reshape(EL, 1)
        x_conv = q + b_rel + out @ W_root
        m = jax.ops.segment_max(x_conv, seg, num_segments=N_GRAPHS)
        x_exp = jnp.exp(x_conv - m[seg])
        denom = jax.ops.segment_sum(x_exp, seg, num_segments=N_GRAPHS)
        scores_e = x_exp / (denom[seg] + 1e-16)
        gx = jax.ops.segment_sum(out * scores_e, seg, num_segments=N_GRAPHS)
        out_list.append(out)
        gout_list.append(jnp.tanh(gx @ W_gout + b_gout))

    gout_all = jnp.stack(gout_list, axis=-1)
    out_all = jnp.stack(out_list, axis=-1)
    sc = jnp.sum(gout_all * a, axis=1, keepdims=True) + a_bias
    sc = jax.nn.softmax(sc, axis=-1)
    sc_edge = sc[seg]
    out_final = jnp.sum(out_all * sc_edge, axis=-1)
    x_new = x + jax.ops.segment_sum(out_final, edge_index[1], num_segments=x.shape[0])
    return x_new


# speedup vs baseline: 3.6339x; 3.4429x over previous
import functools

import jax
import jax.numpy as jnp
from jax import lax
from jax.experimental import pallas as pl
from jax.experimental.pallas import tpu as pltpu
from jax.experimental.pallas import tpu_sc as plsc

NG = 128
G = 128
D = 128
M = 2000


def _mp_body(C, NBH, R, table, ea, srcp, ldstp, offs, outn,
             offs_v, sidx, lraw, rows, acc, sem):
    c = lax.axis_index("c")
    s = lax.axis_index("s")
    pltpu.sync_copy(offs, offs_v)
    pltpu.sync_copy(ea.at[pl.ds(c * C + s * R, R)], acc.at[pl.ds(s * R, R)])
    plsc.subcore_barrier()

    def blk(i, carry):
        b = c + 2 * i
        lo = offs_v[b][0]
        hi = offs_v[b + 1][0]
        n = hi - lo
        K = lax.div(n + 15, jnp.int32(16))
        start = lo + s * K
        end = jnp.minimum(start + K, hi)
        abase = start - lax.rem(start, jnp.int32(8))
        nch = jnp.where(end > start,
                        lax.div(end - abase + (G - 1), jnp.int32(G)),
                        jnp.int32(0))

        def ch(j, _):
            w = pl.multiple_of(abase + j * G, 8)
            pltpu.sync_copy(srcp.at[pl.ds(w, G)], sidx)
            pltpu.sync_copy(ldstp.at[pl.ds(w, G)], lraw)
            pltpu.async_copy(table.at[sidx], rows, sem).wait()
            for k in range(G // 16):
                gi = w + 16 * k + lax.iota(jnp.int32, 16)
                lv = lraw[pl.ds(16 * k, 16)]
                ok = (gi >= start) & (gi < end)
                lfixv = jnp.where(ok, lv, jnp.int32(C))
                pltpu.sync_copy(rows.at[pl.ds(16 * k, 16)], acc.at[lfixv],
                                add=True)
            return 0

        lax.fori_loop(0, nch, ch, 0)
        plsc.subcore_barrier()
        pltpu.sync_copy(acc.at[pl.ds(s * R, R)], outn.at[pl.ds(b * C + s * R, R)])

        @pl.when(i < NBH - 1)
        def _():
            pltpu.sync_copy(ea.at[pl.ds((b + 2) * C + s * R, R)],
                            acc.at[pl.ds(s * R, R)])

        plsc.subcore_barrier()
        return carry

    lax.fori_loop(0, NBH, blk, 0)


def _make_mp_kernel(EL, C):
    NB = EL // C
    NBH = NB // 2
    R = C // 16
    mesh = plsc.VectorSubcoreMesh(core_axis_name="c", subcore_axis_name="s")
    body = functools.partial(_mp_body, C, NBH, R)
    return pl.kernel(
        body,
        out_type=jax.ShapeDtypeStruct((EL, D), jnp.float32),
        scratch_types=[
            pltpu.VMEM((64, 16), jnp.int32),
            pltpu.VMEM((G,), jnp.int32),
            pltpu.VMEM((G,), jnp.int32),
            pltpu.VMEM((G, D), jnp.float32),
            pltpu.VMEM_SHARED((C + 8, D), jnp.float32),
            pltpu.SemaphoreType.DMA,
        ],
        mesh=mesh,
    )


def _q_body(EH, R1, p, srcp, ldstp, offs, qout,
            offs_v, sidx, lraw, prows, zbuf, qacc, sem):
    c = lax.axis_index("c")
    s = lax.axis_index("s")
    pltpu.sync_copy(offs, offs_v)
    for t in range(125):
        zbuf[pl.ds(16 * t, 16)] = jnp.zeros((16,), jnp.float32)
    for t in range(5):
        pltpu.sync_copy(zbuf, qacc.at[pl.ds(s * R1 + 2000 * t, 2000)])
    plsc.subcore_barrier()

    lo = offs_v[c][0]
    hi = offs_v[c + 1][0]
    n = hi - lo
    K = lax.div(n + 15, jnp.int32(16))
    start = lo + s * K
    end = jnp.minimum(start + K, hi)
    abase = start - lax.rem(start, jnp.int32(8))
    nch = jnp.where(end > start,
                    lax.div(end - abase + (G - 1), jnp.int32(G)),
                    jnp.int32(0))

    def ch(j, _):
        w = pl.multiple_of(abase + j * G, 8)
        pltpu.sync_copy(srcp.at[pl.ds(w, G)], sidx)
        pltpu.sync_copy(ldstp.at[pl.ds(w, G)], lraw)
        pltpu.async_copy(p.at[sidx], prows, sem).wait()
        for k in range(G // 16):
            gi = w + 16 * k + lax.iota(jnp.int32, 16)
            lv = lraw[pl.ds(16 * k, 16)]
            ok = (gi >= start) & (gi < end)
            lfixv = jnp.where(ok, lv, jnp.int32(EH))
            pltpu.sync_copy(prows.at[pl.ds(16 * k, 16)], qacc.at[lfixv],
                            add=True)
        return 0

    lax.fori_loop(0, nch, ch, 0)
    plsc.subcore_barrier()
    for t in range(5):
        pltpu.sync_copy(qacc.at[pl.ds(s * R1 + 2000 * t, 2000)], zbuf)
        pltpu.sync_copy(zbuf, qout.at[pl.ds(c * EH + s * R1 + 2000 * t, 2000)])


def _make_q_kernel(EL, EH):
    R1 = EH // 16
    mesh = plsc.VectorSubcoreMesh(core_axis_name="c", subcore_axis_name="s")
    body = functools.partial(_q_body, EH, R1)
    return pl.kernel(
        body,
        out_type=jax.ShapeDtypeStruct((EL,), jnp.float32),
        scratch_types=[
            pltpu.VMEM((16, 16), jnp.int32),
            pltpu.VMEM((G,), jnp.int32),
            pltpu.VMEM((G,), jnp.int32),
            pltpu.VMEM((G,), jnp.float32),
            pltpu.VMEM((2000,), jnp.float32),
            pltpu.VMEM_SHARED((EH + 8,), jnp.float32),
            pltpu.SemaphoreType.DMA,
        ],
        mesh=mesh,
    )


def _ea_body(xu, xv, ecp, ei0p, ei1p, eaout,
             i0v, i1v, ru, rv, rc, comb, sem0, sem1):
    c = lax.axis_index("c")
    s = lax.axis_index("s")
    wid = s * 2 + c
    third = jnp.float32(1.0 / 3.0)

    def _do_chunk(base, nrows):
        pltpu.sync_copy(ei0p.at[pl.ds(base, nrows)], i0v.at[pl.ds(0, nrows)])
        pltpu.sync_copy(ei1p.at[pl.ds(base, nrows)], i1v.at[pl.ds(0, nrows)])
        cp0 = pltpu.async_copy(xu.at[i0v], ru, sem0)
        cp1 = pltpu.async_copy(xv.at[i1v], rv, sem1)
        pltpu.sync_copy(ecp.at[pl.ds(base, nrows)], rc.at[pl.ds(0, nrows)])
        cp0.wait()
        cp1.wait()

        def row(r_, __):
            for k in range(D // 16):
                sl = pl.ds(16 * k, 16)
                comb[r_, sl] = (ru[r_, sl] + rv[r_, sl] + rc[r_, sl]) * third
            return 0

        lax.fori_loop(0, nrows, row, 0)
        pltpu.sync_copy(comb.at[pl.ds(0, nrows)], eaout.at[pl.ds(base, nrows)])

    def ch(j, _):
        base = pl.multiple_of(wid * 10000 + j * G, 8)
        _do_chunk(base, G)
        return 0

    lax.fori_loop(0, 78, ch, 0)
    _do_chunk(pl.multiple_of(wid * 10000 + 78 * G, 8), 16)


def _make_ea_kernel(EL):
    mesh = plsc.VectorSubcoreMesh(core_axis_name="c", subcore_axis_name="s")
    return pl.kernel(
        _ea_body,
        out_type=jax.ShapeDtypeStruct((EL, D), jnp.float32),
        scratch_types=[
            pltpu.VMEM((G,), jnp.int32),
            pltpu.VMEM((G,), jnp.int32),
            pltpu.VMEM((G, D), jnp.float32),
            pltpu.VMEM((G, D), jnp.float32),
            pltpu.VMEM((G, D), jnp.float32),
            pltpu.VMEM((G, D), jnp.float32),
            pltpu.SemaphoreType.DMA,
            pltpu.SemaphoreType.DMA,
        ],
        mesh=mesh,
    )


def _fs_body(EH, NR, of, ei1, zn, npart, nidx, rows, nacc, sem):
    c = lax.axis_index("c")
    s = lax.axis_index("s")
    @pl.when(s < 15)
    def _():
        pltpu.sync_copy(zn.at[pl.ds(s * 640, 640)], nacc.at[pl.ds(s * 640, 640)])

    @pl.when(s == 15)
    def _():
        pltpu.sync_copy(zn.at[pl.ds(9600, NR - 9600)],
                        nacc.at[pl.ds(9600, NR - 9600)])

    plsc.subcore_barrier()
    G2 = 80

    def ch(j, _):
        base = pl.multiple_of(c * EH + s * 10000 + j * G2, 8)
        pltpu.sync_copy(of.at[pl.ds(base, G2)], rows)
        pltpu.sync_copy(ei1.at[pl.ds(base, G2)], nidx)
        for k in range(G2 // 16):
            nidxv = nidx[pl.ds(16 * k, 16)]
            pltpu.sync_copy(rows.at[pl.ds(16 * k, 16)], nacc.at[nidxv],
                            add=True)
        return 0

    lax.fori_loop(0, 125, ch, 0)
    plsc.subcore_barrier()

    @pl.when(s < 15)
    def _():
        pltpu.sync_copy(nacc.at[pl.ds(s * 640, 640)],
                        npart.at[pl.ds(c * NR + s * 640, 640)])

    @pl.when(s == 15)
    def _():
        pltpu.sync_copy(nacc.at[pl.ds(9600, NR - 9600)],
                        npart.at[pl.ds(c * NR + 9600, NR - 9600)])


def _make_fs_kernel(EL, NR):
    EH = EL // 2
    mesh = plsc.VectorSubcoreMesh(core_axis_name="c", subcore_axis_name="s")
    body = functools.partial(_fs_body, EH, NR)
    return pl.kernel(
        body,
        out_type=jax.ShapeDtypeStruct((2 * NR, D), jnp.float32),
        scratch_types=[
            pltpu.VMEM((80,), jnp.int32),
            pltpu.VMEM((80, D), jnp.float32),
            pltpu.VMEM_SHARED((NR, D), jnp.float32),
            pltpu.SemaphoreType.DMA,
        ],
        mesh=mesh,
    )


def _xuv_body(wu, wv, x, xu, xv):
    xb = x[...]
    xu[...] = jnp.dot(xb, wu[...], preferred_element_type=jnp.float32)
    xv[...] = jnp.dot(xb, wv[...], preferred_element_type=jnp.float32)


def _ec_body(we, eattr, ec):
    ec[...] = jnp.dot(eattr[...], we[...], preferred_element_type=jnp.float32)


def _pr_body(wrel, wroot, o, p, r):
    blk = o[...]
    p[...] = jnp.dot(blk, wrel[...], preferred_element_type=jnp.float32)
    r[...] = jnp.dot(blk, wroot[...], preferred_element_type=jnp.float32)


def _m_body(q, r, seg, m, acc):
    j = pl.program_id(0)

    @pl.when(j == 0)
    def _():
        acc[...] = jnp.full((1, NG), -3e38, jnp.float32)

    xc = q[...] + r[...]
    oh = seg[...] == lax.broadcasted_iota(jnp.int32, (1, NG), 1)
    masked = jnp.where(oh, xc, -3e38)
    acc[...] = jnp.maximum(acc[...], jnp.max(masked, axis=0, keepdims=True))

    @pl.when(j == pl.num_programs(0) - 1)
    def _():
        m[...] = acc[...]


def _gx_body(m, wg, bg, an, q, r, seg, o, logit, accgx, accden):
    j = pl.program_id(0)

    @pl.when(j == 0)
    def _():
        accgx[...] = jnp.zeros((NG, NG), jnp.float32)
        accden[...] = jnp.zeros((NG, 1), jnp.float32)

    xc = q[...] + r[...]
    ohf = (seg[...] == lax.broadcasted_iota(jnp.int32, (1, NG), 1)
           ).astype(jnp.float32)
    mg = jnp.dot(ohf, m[...], preferred_element_type=jnp.float32)
    ex = jnp.exp(xc - mg)
    ohw = ohf * ex
    accgx[...] += lax.dot_general(ohw, o[...], (((0,), (0,)), ((), ())),
                                  preferred_element_type=jnp.float32)
    accden[...] += lax.dot_general(ohw, jnp.ones((M, 1), jnp.float32),
                                   (((0,), (0,)), ((), ())),
                                   preferred_element_type=jnp.float32)

    @pl.when(j == pl.num_programs(0) - 1)
    def _():
        gx = accgx[...] / (accden[...] + 1e-16)
        gout = jnp.tanh(jnp.dot(gx, wg[...],
                                preferred_element_type=jnp.float32) + bg[...])
        logit[...] = jnp.dot(gout, an[...], preferred_element_type=jnp.float32)


def _comb_body(lg, ab, seg, o1, o2, o3, o4, of):
    z = lg[...] + ab[...]
    z = z - jnp.max(z, axis=-1, keepdims=True)
    ez = jnp.exp(z)
    sc = ez / jnp.sum(ez, axis=-1, keepdims=True)
    ohf = (seg[...] == lax.broadcasted_iota(jnp.int32, (1, NG), 1)
           ).astype(jnp.float32)
    w = jnp.dot(ohf, sc, preferred_element_type=jnp.float32)
    of[...] = (w[:, 0:1] * o1[...] + w[:, 1:2] * o2[...]
               + w[:, 2:3] * o3[...] + w[:, 3:4] * o4[...])


def _xnew_body(x, n0, n1, xo):
    xo[...] = x[...] + n0[...] + n1[...]


def _blk(shape, imap):
    return pl.BlockSpec(shape, imap)


def _full(*_):
    return (0, 0)


def _rowj(j):
    return (j, 0)


def kernel(x, edge_index, edge_attr, edge_index_bond, edge_index_batch,
           W_u, W_v, W_e, W_rel, b_rel, W_root, a, W_gout, b_gout, a_bias):
    EL = edge_attr.shape[0]
    N = x.shape[0]
    src, dst = edge_index_bond[0], edge_index_bond[1]
    n_iter = a.shape[-1]

    C = 6400
    NB = EL // C
    EH = EL // 2
    EP2 = EL + G
    NR = 10016

    f32 = jnp.float32

    dst_s, src_s = lax.sort_key_val(dst.astype(jnp.int32), src.astype(jnp.int32))
    ldst = (dst_s % C).astype(jnp.int32)
    ldst1 = (dst_s % EH).astype(jnp.int32)
    src_pad = jnp.concatenate([src_s, jnp.zeros((G,), jnp.int32)])
    ldst_pad = jnp.concatenate([ldst, jnp.full((G,), C, jnp.int32)])
    ldst1_pad = jnp.concatenate([ldst1, jnp.full((G,), EH, jnp.int32)])
    offs = jnp.searchsorted(dst_s, jnp.arange(NB + 1, dtype=jnp.int32) * C)
    offs = jnp.concatenate([offs.astype(jnp.int32),
                            jnp.zeros((64 - NB - 1,), jnp.int32)])
    offs = jnp.tile(offs[:, None], (1, 16))
    qoffs = jnp.searchsorted(dst_s, jnp.arange(3, dtype=jnp.int32) * EH)
    qoffs = jnp.concatenate([qoffs.astype(jnp.int32),
                             jnp.zeros((13,), jnp.int32)])
    qoffs = jnp.tile(qoffs[:, None], (1, 16))

    ei0 = edge_index[0].astype(jnp.int32)
    ei1 = edge_index[1].astype(jnp.int32)
    seg2 = edge_index_batch.astype(jnp.int32).reshape(EL, 1)
    zn = jnp.zeros((NR, D), f32)

    mp = _make_mp_kernel(EL, C)
    qk = _make_q_kernel(EL, EH)
    eak = _make_ea_kernel(EL)
    fsk = _make_fs_kernel(EL, NR)

    NBLK = EL // M

    xu, xv = pl.pallas_call(
        _xuv_body,
        grid=(N // 1000,),
        in_specs=[_blk((D, D), _full), _blk((D, D), _full),
                  _blk((1000, D), _rowj)],
        out_specs=[_blk((1000, D), _rowj)] * 2,
        out_shape=[jax.ShapeDtypeStruct((N, D), f32)] * 2,
    )(W_u, W_v, x)

    ec = pl.pallas_call(
        _ec_body,
        grid=(NBLK,),
        in_specs=[_blk((16, D), _full), _blk((M, 16), _rowj)],
        out_specs=_blk((M, D), _rowj),
        out_shape=jax.ShapeDtypeStruct((EL, D), f32),
    )(W_e, edge_attr)

    ea = eak(xu, xv, ec, ei0, ei1)

    wrel = W_rel.astype(f32)
    wroot = W_root.astype(f32)
    bg = b_gout.reshape(1, D).astype(f32)
    ab = a_bias.reshape(1, n_iter).astype(f32)

    out = ea
    outs = []
    logits = []
    for n in range(n_iter):
        out = mp(out, ea, src_pad, ldst_pad, offs)
        p2, r2 = pl.pallas_call(
            _pr_body,
            grid=(NBLK,),
            in_specs=[_blk((D, 1), _full), _blk((D, 1), _full),
                      _blk((M, D), _rowj)],
            out_specs=[_blk((M, 1), _rowj)] * 2,
            out_shape=[jax.ShapeDtypeStruct((EL, 1), f32)] * 2,
        )(wrel, wroot, out)
        q = qk(p2.reshape(EL), src_pad, ldst1_pad, qoffs)
        q2 = q.reshape(EL, 1)
        m = pl.pallas_call(
            _m_body,
            grid=(NBLK,),
            in_specs=[_blk((M, 1), _rowj), _blk((M, 1), _rowj),
                      _blk((M, 1), _rowj)],
            out_specs=_blk((1, NG), _full),
            out_shape=jax.ShapeDtypeStruct((1, NG), f32),
            scratch_shapes=[pltpu.VMEM((1, NG), f32)],
        )(q2, r2, seg2)
        logit = pl.pallas_call(
            _gx_body,
            grid=(NBLK,),
            in_specs=[_blk((NG, 1), _full), _blk((D, D), _full),
                      _blk((1, D), _full), _blk((D, 1), _full),
                      _blk((M, 1), _rowj), _blk((M, 1), _rowj),
                      _blk((M, 1), _rowj), _blk((M, D), _rowj)],
            out_specs=_blk((NG, 1), _full),
            out_shape=jax.ShapeDtypeStruct((NG, 1), f32),
            scratch_shapes=[pltpu.VMEM((NG, NG), f32),
                            pltpu.VMEM((NG, 1), f32)],
        )(m.reshape(NG, 1), W_gout.astype(f32), bg,
          a[0, :, n].reshape(D, 1).astype(f32), q2, r2, seg2, out)
        outs.append(out)
        logits.append(logit)

    lg = jnp.concatenate(logits, axis=1)

    out_final = pl.pallas_call(
        _comb_body,
        grid=(NBLK,),
        in_specs=[_blk((NG, n_iter), _full), _blk((1, n_iter), _full),
                  _blk((M, 1), _rowj), _blk((M, D), _rowj),
                  _blk((M, D), _rowj), _blk((M, D), _rowj),
                  _blk((M, D), _rowj)],
        out_specs=_blk((M, D), _rowj),
        out_shape=jax.ShapeDtypeStruct((EL, D), f32),
    )(lg, ab, seg2, *outs)

    npart = fsk(out_final, ei1, zn)

    x_new = pl.pallas_call(
        _xnew_body,
        grid=(N // 1000,),
        in_specs=[_blk((1000, D), _rowj), _blk((1000, D), _rowj),
                  _blk((1000, D), _rowj)],
        out_specs=_blk((1000, D), _rowj),
        out_shape=jax.ShapeDtypeStruct((N, D), f32),
    )(x, npart[:N], npart[NR:NR + N])
    return x_new


# speedup vs baseline: 3.8133x; 1.0494x over previous
import functools

import jax
import jax.numpy as jnp
from jax import lax
from jax.experimental import pallas as pl
from jax.experimental.pallas import tpu as pltpu
from jax.experimental.pallas import tpu_sc as plsc

NG = 128
G = 128
D = 128
M = 2000


def _mp_body(C, NBH, R, table, ea, srcp, ldstp, offs, outn,
             offs_v, sidx, lraw, rows, acc, sem, sem2):
    c = lax.axis_index("c")
    s = lax.axis_index("s")
    pltpu.sync_copy(offs, offs_v)
    pltpu.sync_copy(ea.at[pl.ds(c * C + s * R, R)], acc.at[pl.ds(s * R, R)])
    plsc.subcore_barrier()

    def blk(i, carry):
        b = c + 2 * i
        lo = offs_v[b][0]
        hi = offs_v[b + 1][0]
        n = hi - lo
        K = lax.div(n + 15, jnp.int32(16))
        start = lo + s * K
        end = jnp.minimum(start + K, hi)
        abase = start - lax.rem(start, jnp.int32(8))
        nch = jnp.where(end > start,
                        lax.div(end - abase + (G - 1), jnp.int32(G)),
                        jnp.int32(0))

        def ch(j, _):
            w = pl.multiple_of(abase + j * G, 8)
            ci = pltpu.async_copy(srcp.at[pl.ds(w, G)], sidx, sem)
            cl = pltpu.async_copy(ldstp.at[pl.ds(w, G)], lraw, sem2)
            ci.wait()
            cl.wait()
            pltpu.async_copy(table.at[sidx], rows, sem).wait()
            interior = (w >= start) & (w + G <= end)

            @pl.when(interior)
            def _():
                pltpu.sync_copy(rows, acc.at[lraw], add=True)

            @pl.when(jnp.logical_not(interior))
            def _():
                ds = []
                for k in range(G // 16):
                    gi = w + 16 * k + lax.iota(jnp.int32, 16)
                    lv = lraw[pl.ds(16 * k, 16)]
                    ok = (gi >= start) & (gi < end)
                    lfixv = jnp.where(ok, lv, jnp.int32(C))
                    ds.append(pltpu.async_copy(rows.at[pl.ds(16 * k, 16)],
                                               acc.at[lfixv], sem2, add=True))
                for d in ds:
                    d.wait()

            return 0

        lax.fori_loop(0, nch, ch, 0)
        plsc.subcore_barrier()
        pltpu.sync_copy(acc.at[pl.ds(s * R, R)], outn.at[pl.ds(b * C + s * R, R)])

        @pl.when(i < NBH - 1)
        def _():
            pltpu.sync_copy(ea.at[pl.ds((b + 2) * C + s * R, R)],
                            acc.at[pl.ds(s * R, R)])

        plsc.subcore_barrier()
        return carry

    lax.fori_loop(0, NBH, blk, 0)


def _make_mp_kernel(EL, C):
    NB = EL // C
    NBH = NB // 2
    R = C // 16
    mesh = plsc.VectorSubcoreMesh(core_axis_name="c", subcore_axis_name="s")
    body = functools.partial(_mp_body, C, NBH, R)
    return pl.kernel(
        body,
        out_type=jax.ShapeDtypeStruct((EL, D), jnp.float32),
        scratch_types=[
            pltpu.VMEM((64, 16), jnp.int32),
            pltpu.VMEM((G,), jnp.int32),
            pltpu.VMEM((G,), jnp.int32),
            pltpu.VMEM((G, D), jnp.float32),
            pltpu.VMEM_SHARED((C + 8, D), jnp.float32),
            pltpu.SemaphoreType.DMA,
            pltpu.SemaphoreType.DMA,
        ],
        mesh=mesh,
    )


def _q_body(EH, R1, p, srcp, ldstp, offs, qout,
            offs_v, sidx, lraw, prows, zbuf, qacc, sem):
    c = lax.axis_index("c")
    s = lax.axis_index("s")
    pltpu.sync_copy(offs, offs_v)
    for t in range(125):
        zbuf[pl.ds(16 * t, 16)] = jnp.zeros((16,), jnp.float32)
    for t in range(5):
        pltpu.sync_copy(zbuf, qacc.at[pl.ds(s * R1 + 2000 * t, 2000)])
    plsc.subcore_barrier()

    lo = offs_v[c][0]
    hi = offs_v[c + 1][0]
    n = hi - lo
    K = lax.div(n + 15, jnp.int32(16))
    start = lo + s * K
    end = jnp.minimum(start + K, hi)
    abase = start - lax.rem(start, jnp.int32(8))
    nch = jnp.where(end > start,
                    lax.div(end - abase + (G - 1), jnp.int32(G)),
                    jnp.int32(0))

    def ch(j, _):
        w = pl.multiple_of(abase + j * G, 8)
        pltpu.sync_copy(srcp.at[pl.ds(w, G)], sidx)
        pltpu.sync_copy(ldstp.at[pl.ds(w, G)], lraw)
        pltpu.async_copy(p.at[sidx], prows, sem).wait()
        for k in range(G // 16):
            gi = w + 16 * k + lax.iota(jnp.int32, 16)
            lv = lraw[pl.ds(16 * k, 16)]
            ok = (gi >= start) & (gi < end)
            lfixv = jnp.where(ok, lv, jnp.int32(EH))
            pltpu.sync_copy(prows.at[pl.ds(16 * k, 16)], qacc.at[lfixv],
                            add=True)
        return 0

    lax.fori_loop(0, nch, ch, 0)
    plsc.subcore_barrier()
    for t in range(5):
        pltpu.sync_copy(qacc.at[pl.ds(s * R1 + 2000 * t, 2000)], zbuf)
        pltpu.sync_copy(zbuf, qout.at[pl.ds(c * EH + s * R1 + 2000 * t, 2000)])


def _make_q_kernel(EL, EH):
    R1 = EH // 16
    mesh = plsc.VectorSubcoreMesh(core_axis_name="c", subcore_axis_name="s")
    body = functools.partial(_q_body, EH, R1)
    return pl.kernel(
        body,
        out_type=jax.ShapeDtypeStruct((EL,), jnp.float32),
        scratch_types=[
            pltpu.VMEM((16, 16), jnp.int32),
            pltpu.VMEM((G,), jnp.int32),
            pltpu.VMEM((G,), jnp.int32),
            pltpu.VMEM((G,), jnp.float32),
            pltpu.VMEM((2000,), jnp.float32),
            pltpu.VMEM_SHARED((EH + 8,), jnp.float32),
            pltpu.SemaphoreType.DMA,
        ],
        mesh=mesh,
    )


def _ea_body(xu, xv, ecp, ei0p, ei1p, eaout,
             i0v, i1v, ru, rv, rc, comb, sem0, sem1):
    c = lax.axis_index("c")
    s = lax.axis_index("s")
    wid = s * 2 + c
    third = jnp.float32(1.0 / 3.0)

    def _do_chunk(base, nrows):
        pltpu.sync_copy(ei0p.at[pl.ds(base, nrows)], i0v.at[pl.ds(0, nrows)])
        pltpu.sync_copy(ei1p.at[pl.ds(base, nrows)], i1v.at[pl.ds(0, nrows)])
        cp0 = pltpu.async_copy(xu.at[i0v], ru, sem0)
        cp1 = pltpu.async_copy(xv.at[i1v], rv, sem1)
        pltpu.sync_copy(ecp.at[pl.ds(base, nrows)], rc.at[pl.ds(0, nrows)])
        cp0.wait()
        cp1.wait()

        def row(r_, __):
            for k in range(D // 16):
                sl = pl.ds(16 * k, 16)
                comb[r_, sl] = (ru[r_, sl] + rv[r_, sl] + rc[r_, sl]) * third
            return 0

        lax.fori_loop(0, nrows, row, 0)
        pltpu.sync_copy(comb.at[pl.ds(0, nrows)], eaout.at[pl.ds(base, nrows)])

    def ch(j, _):
        base = pl.multiple_of(wid * 10000 + j * G, 8)
        _do_chunk(base, G)
        return 0

    lax.fori_loop(0, 78, ch, 0)
    _do_chunk(pl.multiple_of(wid * 10000 + 78 * G, 8), 16)


def _make_ea_kernel(EL):
    mesh = plsc.VectorSubcoreMesh(core_axis_name="c", subcore_axis_name="s")
    return pl.kernel(
        _ea_body,
        out_type=jax.ShapeDtypeStruct((EL, D), jnp.float32),
        scratch_types=[
            pltpu.VMEM((G,), jnp.int32),
            pltpu.VMEM((G,), jnp.int32),
            pltpu.VMEM((G, D), jnp.float32),
            pltpu.VMEM((G, D), jnp.float32),
            pltpu.VMEM((G, D), jnp.float32),
            pltpu.VMEM((G, D), jnp.float32),
            pltpu.SemaphoreType.DMA,
            pltpu.SemaphoreType.DMA,
        ],
        mesh=mesh,
    )


def _fs_body(EH, NR, of, ei1, zn, npart, nidx, rows, nacc, sem):
    c = lax.axis_index("c")
    s = lax.axis_index("s")
    @pl.when(s < 15)
    def _():
        pltpu.sync_copy(zn.at[pl.ds(s * 640, 640)], nacc.at[pl.ds(s * 640, 640)])

    @pl.when(s == 15)
    def _():
        pltpu.sync_copy(zn.at[pl.ds(9600, NR - 9600)],
                        nacc.at[pl.ds(9600, NR - 9600)])

    plsc.subcore_barrier()
    G2 = 80

    def ch(j, _):
        base = pl.multiple_of(c * EH + s * 10000 + j * G2, 8)
        cr = pltpu.async_copy(of.at[pl.ds(base, G2)], rows, sem)
        pltpu.sync_copy(ei1.at[pl.ds(base, G2)], nidx)
        cr.wait()
        pltpu.sync_copy(rows, nacc.at[nidx], add=True)
        return 0

    lax.fori_loop(0, 125, ch, 0)
    plsc.subcore_barrier()

    @pl.when(s < 15)
    def _():
        pltpu.sync_copy(nacc.at[pl.ds(s * 640, 640)],
                        npart.at[pl.ds(c * NR + s * 640, 640)])

    @pl.when(s == 15)
    def _():
        pltpu.sync_copy(nacc.at[pl.ds(9600, NR - 9600)],
                        npart.at[pl.ds(c * NR + 9600, NR - 9600)])


def _make_fs_kernel(EL, NR):
    EH = EL // 2
    mesh = plsc.VectorSubcoreMesh(core_axis_name="c", subcore_axis_name="s")
    body = functools.partial(_fs_body, EH, NR)
    return pl.kernel(
        body,
        out_type=jax.ShapeDtypeStruct((2 * NR, D), jnp.float32),
        scratch_types=[
            pltpu.VMEM((80,), jnp.int32),
            pltpu.VMEM((80, D), jnp.float32),
            pltpu.VMEM_SHARED((NR, D), jnp.float32),
            pltpu.SemaphoreType.DMA,
        ],
        mesh=mesh,
    )


def _xuv_body(wu, wv, x, xu, xv):
    xb = x[...]
    xu[...] = jnp.dot(xb, wu[...], preferred_element_type=jnp.float32)
    xv[...] = jnp.dot(xb, wv[...], preferred_element_type=jnp.float32)


def _ec_body(we, eattr, ec):
    ec[...] = jnp.dot(eattr[...], we[...], preferred_element_type=jnp.float32)


def _pr_body(wrel, wroot, o, p, r):
    blk = o[...]
    p[...] = jnp.dot(blk, wrel[...], preferred_element_type=jnp.float32)
    r[...] = jnp.dot(blk, wroot[...], preferred_element_type=jnp.float32)


def _m_body(q, r, seg, m, acc):
    j = pl.program_id(0)

    @pl.when(j == 0)
    def _():
        acc[...] = jnp.full((1, NG), -3e38, jnp.float32)

    xc = q[...] + r[...]
    oh = seg[...] == lax.broadcasted_iota(jnp.int32, (1, NG), 1)
    masked = jnp.where(oh, xc, -3e38)
    acc[...] = jnp.maximum(acc[...], jnp.max(masked, axis=0, keepdims=True))

    @pl.when(j == pl.num_programs(0) - 1)
    def _():
        m[...] = acc[...]


def _gx_body(m, wg, bg, an, q, r, seg, o, logit, accgx, accden):
    j = pl.program_id(0)

    @pl.when(j == 0)
    def _():
        accgx[...] = jnp.zeros((NG, NG), jnp.float32)
        accden[...] = jnp.zeros((NG, 1), jnp.float32)

    xc = q[...] + r[...]
    ohf = (seg[...] == lax.broadcasted_iota(jnp.int32, (1, NG), 1)
           ).astype(jnp.float32)
    mg = jnp.dot(ohf, m[...], preferred_element_type=jnp.float32)
    ex = jnp.exp(xc - mg)
    ohw = ohf * ex
    accgx[...] += lax.dot_general(ohw, o[...], (((0,), (0,)), ((), ())),
                                  preferred_element_type=jnp.float32)
    accden[...] += lax.dot_general(ohw, jnp.ones((M, 1), jnp.float32),
                                   (((0,), (0,)), ((), ())),
                                   preferred_element_type=jnp.float32)

    @pl.when(j == pl.num_programs(0) - 1)
    def _():
        gx = accgx[...] / (accden[...] + 1e-16)
        gout = jnp.tanh(jnp.dot(gx, wg[...],
                                preferred_element_type=jnp.float32) + bg[...])
        logit[...] = jnp.dot(gout, an[...], preferred_element_type=jnp.float32)


def _comb_body(lg, ab, seg, o1, o2, o3, o4, of):
    z = lg[...] + ab[...]
    z = z - jnp.max(z, axis=-1, keepdims=True)
    ez = jnp.exp(z)
    sc = ez / jnp.sum(ez, axis=-1, keepdims=True)
    ohf = (seg[...] == lax.broadcasted_iota(jnp.int32, (1, NG), 1)
           ).astype(jnp.float32)
    w = jnp.dot(ohf, sc, preferred_element_type=jnp.float32)
    of[...] = (w[:, 0:1] * o1[...] + w[:, 1:2] * o2[...]
               + w[:, 2:3] * o3[...] + w[:, 3:4] * o4[...])


def _xnew_body(x, n0, n1, xo):
    xo[...] = x[...] + n0[...] + n1[...]


def _blk(shape, imap):
    return pl.BlockSpec(shape, imap)


def _full(*_):
    return (0, 0)


def _rowj(j):
    return (j, 0)


def kernel(x, edge_index, edge_attr, edge_index_bond, edge_index_batch,
           W_u, W_v, W_e, W_rel, b_rel, W_root, a, W_gout, b_gout, a_bias):
    EL = edge_attr.shape[0]
    N = x.shape[0]
    src, dst = edge_index_bond[0], edge_index_bond[1]
    n_iter = a.shape[-1]

    C = 6400
    NB = EL // C
    EH = EL // 2
    EP2 = EL + G
    NR = 10016

    f32 = jnp.float32

    dst_s, src_s = lax.sort_key_val(dst.astype(jnp.int32), src.astype(jnp.int32))
    ldst = (dst_s % C).astype(jnp.int32)
    ldst1 = (dst_s % EH).astype(jnp.int32)
    src_pad = jnp.concatenate([src_s, jnp.zeros((G,), jnp.int32)])
    ldst_pad = jnp.concatenate([ldst, jnp.full((G,), C, jnp.int32)])
    ldst1_pad = jnp.concatenate([ldst1, jnp.full((G,), EH, jnp.int32)])
    offs = jnp.searchsorted(dst_s, jnp.arange(NB + 1, dtype=jnp.int32) * C)
    offs = jnp.concatenate([offs.astype(jnp.int32),
                            jnp.zeros((64 - NB - 1,), jnp.int32)])
    offs = jnp.tile(offs[:, None], (1, 16))
    qoffs = jnp.searchsorted(dst_s, jnp.arange(3, dtype=jnp.int32) * EH)
    qoffs = jnp.concatenate([qoffs.astype(jnp.int32),
                             jnp.zeros((13,), jnp.int32)])
    qoffs = jnp.tile(qoffs[:, None], (1, 16))

    ei0 = edge_index[0].astype(jnp.int32)
    ei1 = edge_index[1].astype(jnp.int32)
    seg2 = edge_index_batch.astype(jnp.int32).reshape(EL, 1)
    zn = jnp.zeros((NR, D), f32)

    mp = _make_mp_kernel(EL, C)
    qk = _make_q_kernel(EL, EH)
    eak = _make_ea_kernel(EL)
    fsk = _make_fs_kernel(EL, NR)

    NBLK = EL // M

    xu, xv = pl.pallas_call(
        _xuv_body,
        grid=(N // 1000,),
        in_specs=[_blk((D, D), _full), _blk((D, D), _full),
                  _blk((1000, D), _rowj)],
        out_specs=[_blk((1000, D), _rowj)] * 2,
        out_shape=[jax.ShapeDtypeStruct((N, D), f32)] * 2,
    )(W_u, W_v, x)

    ec = pl.pallas_call(
        _ec_body,
        grid=(NBLK,),
        in_specs=[_blk((16, D), _full), _blk((M, 16), _rowj)],
        out_specs=_blk((M, D), _rowj),
        out_shape=jax.ShapeDtypeStruct((EL, D), f32),
    )(W_e, edge_attr)

    ea = eak(xu, xv, ec, ei0, ei1)

    wrel = W_rel.astype(f32)
    wroot = W_root.astype(f32)
    bg = b_gout.reshape(1, D).astype(f32)
    ab = a_bias.reshape(1, n_iter).astype(f32)

    out = ea
    outs = []
    logits = []
    for n in range(n_iter):
        out = mp(out, ea, src_pad, ldst_pad, offs)
        p2, r2 = pl.pallas_call(
            _pr_body,
            grid=(NBLK,),
            in_specs=[_blk((D, 1), _full), _blk((D, 1), _full),
                      _blk((M, D), _rowj)],
            out_specs=[_blk((M, 1), _rowj)] * 2,
            out_shape=[jax.ShapeDtypeStruct((EL, 1), f32)] * 2,
        )(wrel, wroot, out)
        q = qk(p2.reshape(EL), src_pad, ldst1_pad, qoffs)
        q2 = q.reshape(EL, 1)
        m = pl.pallas_call(
            _m_body,
            grid=(NBLK,),
            in_specs=[_blk((M, 1), _rowj), _blk((M, 1), _rowj),
                      _blk((M, 1), _rowj)],
            out_specs=_blk((1, NG), _full),
            out_shape=jax.ShapeDtypeStruct((1, NG), f32),
            scratch_shapes=[pltpu.VMEM((1, NG), f32)],
        )(q2, r2, seg2)
        logit = pl.pallas_call(
            _gx_body,
            grid=(NBLK,),
            in_specs=[_blk((NG, 1), _full), _blk((D, D), _full),
                      _blk((1, D), _full), _blk((D, 1), _full),
                      _blk((M, 1), _rowj), _blk((M, 1), _rowj),
                      _blk((M, 1), _rowj), _blk((M, D), _rowj)],
            out_specs=_blk((NG, 1), _full),
            out_shape=jax.ShapeDtypeStruct((NG, 1), f32),
            scratch_shapes=[pltpu.VMEM((NG, NG), f32),
                            pltpu.VMEM((NG, 1), f32)],
        )(m.reshape(NG, 1), W_gout.astype(f32), bg,
          a[0, :, n].reshape(D, 1).astype(f32), q2, r2, seg2, out)
        outs.append(out)
        logits.append(logit)

    lg = jnp.concatenate(logits, axis=1)

    out_final = pl.pallas_call(
        _comb_body,
        grid=(NBLK,),
        in_specs=[_blk((NG, n_iter), _full), _blk((1, n_iter), _full),
                  _blk((M, 1), _rowj), _blk((M, D), _rowj),
                  _blk((M, D), _rowj), _blk((M, D), _rowj),
                  _blk((M, D), _rowj)],
        out_specs=_blk((M, D), _rowj),
        out_shape=jax.ShapeDtypeStruct((EL, D), f32),
    )(lg, ab, seg2, *outs)

    npart = fsk(out_final, ei1, zn)

    x_new = pl.pallas_call(
        _xnew_body,
        grid=(N // 1000,),
        in_specs=[_blk((1000, D), _rowj), _blk((1000, D), _rowj),
                  _blk((1000, D), _rowj)],
        out_specs=_blk((1000, D), _rowj),
        out_shape=jax.ShapeDtypeStruct((N, D), f32),
    )(x, npart[:N], npart[NR:NR + N])
    return x_new


# speedup vs baseline: 4.0140x; 1.0526x over previous
import functools

import jax
import jax.numpy as jnp
from jax import lax
from jax.experimental import pallas as pl
from jax.experimental.pallas import tpu as pltpu
from jax.experimental.pallas import tpu_sc as plsc

NG = 128
G = 128
D = 128
M = 2000


def _mp_body(C, CB, NBH, R, table, ea, srcp, ld0, ld1, ld2, offs, outn,
             offs_v, sidx, lraw, rows, acc,
             si0, si1, si2, sw0, sw1, sw2, semg, sems):
    c = lax.axis_index("c")
    s = lax.axis_index("s")
    semi = [si0, si1, si2]
    semw = [sw0, sw1, sw2]
    lds = [ld0, ld1, ld2]
    pltpu.sync_copy(offs, offs_v)
    pltpu.sync_copy(ea.at[pl.ds(c * C + s * R, R)], acc.at[pl.ds(s * R, R)])
    pltpu.async_copy(ea.at[pl.ds((c + 2) * C + s * R, R)],
                     acc.at[pl.ds(CB + s * R, R)], si1)
    plsc.subcore_barrier()

    def scatter_block(i, h):
        b = c + 2 * i
        base = h * CB
        dump = jnp.int32(base + C)
        lo = offs_v[b][0]
        hi = offs_v[b + 1][0]
        n = hi - lo
        K = lax.div(n + 15, jnp.int32(16))
        start = lo + s * K
        end = jnp.minimum(start + K, hi)
        abase = start - lax.rem(start, jnp.int32(8))
        nch = jnp.where(end > start,
                        lax.div(end - abase + (G - 1), jnp.int32(G)),
                        jnp.int32(0))

        def ch(j, _):
            w = pl.multiple_of(abase + j * G, 8)
            ci = pltpu.async_copy(srcp.at[pl.ds(w, G)], sidx, semg)
            cl = pltpu.async_copy(lds[h].at[pl.ds(w, G)], lraw, sems)
            ci.wait()
            cl.wait()
            pltpu.async_copy(table.at[sidx], rows, semg).wait()
            interior = (w >= start) & (w + G <= end)

            @pl.when(interior)
            def _():
                pltpu.sync_copy(rows, acc.at[lraw], add=True)

            @pl.when(jnp.logical_not(interior))
            def _():
                dl = []
                for k in range(G // 16):
                    gi = w + 16 * k + lax.iota(jnp.int32, 16)
                    lv = lraw[pl.ds(16 * k, 16)]
                    ok = (gi >= start) & (gi < end)
                    lfixv = jnp.where(ok, lv, dump)
                    dl.append(pltpu.async_copy(rows.at[pl.ds(16 * k, 16)],
                                               acc.at[lfixv], sems, add=True))
                for d in dl:
                    d.wait()

            return 0

        lax.fori_loop(0, nch, ch, 0)

    def macro(t, carry):
        for h in range(3):
            i = 3 * t + h
            hp = (h + 2) % 3

            @pl.when(i < NBH)
            def _():
                @pl.when(i >= 1)
                def _():
                    pltpu.make_async_copy(
                        ea.at[pl.ds(0, R)],
                        acc.at[pl.ds(h * CB + s * R, R)], semi[h]).wait()

                plsc.subcore_barrier()
                scatter_block(i, h)
                plsc.subcore_barrier()

                @pl.when(i >= 1)
                def _():
                    pltpu.make_async_copy(
                        acc.at[pl.ds(hp * CB + s * R, R)],
                        outn.at[pl.ds(s * R, R)], semw[hp]).wait()

                @pl.when(i + 2 < NBH)
                def _():
                    pltpu.async_copy(
                        ea.at[pl.ds((c + 2 * (i + 2)) * C + s * R, R)],
                        acc.at[pl.ds(hp * CB + s * R, R)], semi[hp])

                b = c + 2 * i
                pltpu.async_copy(acc.at[pl.ds(h * CB + s * R, R)],
                                 outn.at[pl.ds(b * C + s * R, R)], semw[h])

        return carry

    lax.fori_loop(0, (NBH + 2) // 3, macro, 0)
    hl = (NBH - 1) % 3
    pltpu.make_async_copy(acc.at[pl.ds(hl * CB + s * R, R)],
                          outn.at[pl.ds(s * R, R)], semw[hl]).wait()


def _make_mp_kernel(EL, C):
    NB = EL // C
    NBH = NB // 2
    R = C // 16
    CB = C + 16
    mesh = plsc.VectorSubcoreMesh(core_axis_name="c", subcore_axis_name="s")
    body = functools.partial(_mp_body, C, CB, NBH, R)
    return pl.kernel(
        body,
        out_type=jax.ShapeDtypeStruct((EL, D), jnp.float32),
        scratch_types=[
            pltpu.VMEM((112, 16), jnp.int32),
            pltpu.VMEM((G,), jnp.int32),
            pltpu.VMEM((G,), jnp.int32),
            pltpu.VMEM((G, D), jnp.float32),
            pltpu.VMEM_SHARED((3 * CB, D), jnp.float32),
            pltpu.SemaphoreType.DMA,
            pltpu.SemaphoreType.DMA,
            pltpu.SemaphoreType.DMA,
            pltpu.SemaphoreType.DMA,
            pltpu.SemaphoreType.DMA,
            pltpu.SemaphoreType.DMA,
            pltpu.SemaphoreType.DMA,
            pltpu.SemaphoreType.DMA,
        ],
        mesh=mesh,
    )


def _q_body(EH, R1, p, srcp, ldstp, offs, qout,
            offs_v, sidx, lraw, prows, zbuf, qacc, sem):
    c = lax.axis_index("c")
    s = lax.axis_index("s")
    pltpu.sync_copy(offs, offs_v)
    for t in range(125):
        zbuf[pl.ds(16 * t, 16)] = jnp.zeros((16,), jnp.float32)
    for t in range(5):
        pltpu.sync_copy(zbuf, qacc.at[pl.ds(s * R1 + 2000 * t, 2000)])
    plsc.subcore_barrier()

    lo = offs_v[c][0]
    hi = offs_v[c + 1][0]
    n = hi - lo
    K = lax.div(n + 15, jnp.int32(16))
    start = lo + s * K
    end = jnp.minimum(start + K, hi)
    abase = start - lax.rem(start, jnp.int32(8))
    nch = jnp.where(end > start,
                    lax.div(end - abase + (G - 1), jnp.int32(G)),
                    jnp.int32(0))

    def ch(j, _):
        w = pl.multiple_of(abase + j * G, 8)
        pltpu.sync_copy(srcp.at[pl.ds(w, G)], sidx)
        pltpu.sync_copy(ldstp.at[pl.ds(w, G)], lraw)
        pltpu.async_copy(p.at[sidx], prows, sem).wait()
        for k in range(G // 16):
            gi = w + 16 * k + lax.iota(jnp.int32, 16)
            lv = lraw[pl.ds(16 * k, 16)]
            ok = (gi >= start) & (gi < end)
            lfixv = jnp.where(ok, lv, jnp.int32(EH))
            pltpu.sync_copy(prows.at[pl.ds(16 * k, 16)], qacc.at[lfixv],
                            add=True)
        return 0

    lax.fori_loop(0, nch, ch, 0)
    plsc.subcore_barrier()
    for t in range(5):
        pltpu.sync_copy(qacc.at[pl.ds(s * R1 + 2000 * t, 2000)], zbuf)
        pltpu.sync_copy(zbuf, qout.at[pl.ds(c * EH + s * R1 + 2000 * t, 2000)])


def _make_q_kernel(EL, EH):
    R1 = EH // 16
    mesh = plsc.VectorSubcoreMesh(core_axis_name="c", subcore_axis_name="s")
    body = functools.partial(_q_body, EH, R1)
    return pl.kernel(
        body,
        out_type=jax.ShapeDtypeStruct((EL,), jnp.float32),
        scratch_types=[
            pltpu.VMEM((16, 16), jnp.int32),
            pltpu.VMEM((G,), jnp.int32),
            pltpu.VMEM((G,), jnp.int32),
            pltpu.VMEM((G,), jnp.float32),
            pltpu.VMEM((2000,), jnp.float32),
            pltpu.VMEM_SHARED((EH + 8,), jnp.float32),
            pltpu.SemaphoreType.DMA,
        ],
        mesh=mesh,
    )


def _ea_body(xu, xv, ecp, ei0p, ei1p, eaout,
             i0v, i1v, ru, rv, rc, comb, sem0, sem1):
    c = lax.axis_index("c")
    s = lax.axis_index("s")
    wid = s * 2 + c
    third = jnp.float32(1.0 / 3.0)

    def _do_chunk(base, nrows):
        pltpu.sync_copy(ei0p.at[pl.ds(base, nrows)], i0v.at[pl.ds(0, nrows)])
        pltpu.sync_copy(ei1p.at[pl.ds(base, nrows)], i1v.at[pl.ds(0, nrows)])
        cp0 = pltpu.async_copy(xu.at[i0v], ru, sem0)
        cp1 = pltpu.async_copy(xv.at[i1v], rv, sem1)
        pltpu.sync_copy(ecp.at[pl.ds(base, nrows)], rc.at[pl.ds(0, nrows)])
        cp0.wait()
        cp1.wait()

        def row(r_, __):
            for k in range(D // 16):
                sl = pl.ds(16 * k, 16)
                comb[r_, sl] = (ru[r_, sl] + rv[r_, sl] + rc[r_, sl]) * third
            return 0

        lax.fori_loop(0, nrows, row, 0)
        pltpu.sync_copy(comb.at[pl.ds(0, nrows)], eaout.at[pl.ds(base, nrows)])

    def ch(j, _):
        base = pl.multiple_of(wid * 10000 + j * G, 8)
        _do_chunk(base, G)
        return 0

    lax.fori_loop(0, 78, ch, 0)
    _do_chunk(pl.multiple_of(wid * 10000 + 78 * G, 8), 16)


def _make_ea_kernel(EL):
    mesh = plsc.VectorSubcoreMesh(core_axis_name="c", subcore_axis_name="s")
    return pl.kernel(
        _ea_body,
        out_type=jax.ShapeDtypeStruct((EL, D), jnp.float32),
        scratch_types=[
            pltpu.VMEM((G,), jnp.int32),
            pltpu.VMEM((G,), jnp.int32),
            pltpu.VMEM((G, D), jnp.float32),
            pltpu.VMEM((G, D), jnp.float32),
            pltpu.VMEM((G, D), jnp.float32),
            pltpu.VMEM((G, D), jnp.float32),
            pltpu.SemaphoreType.DMA,
            pltpu.SemaphoreType.DMA,
        ],
        mesh=mesh,
    )


def _fs_body(EH, NR, of, ei1, zn, npart, nidx, rows, nacc, sem):
    c = lax.axis_index("c")
    s = lax.axis_index("s")
    @pl.when(s < 15)
    def _():
        pltpu.sync_copy(zn.at[pl.ds(s * 640, 640)], nacc.at[pl.ds(s * 640, 640)])

    @pl.when(s == 15)
    def _():
        pltpu.sync_copy(zn.at[pl.ds(9600, NR - 9600)],
                        nacc.at[pl.ds(9600, NR - 9600)])

    plsc.subcore_barrier()
    G2 = 80

    def ch(j, _):
        base = pl.multiple_of(c * EH + s * 10000 + j * G2, 8)
        cr = pltpu.async_copy(of.at[pl.ds(base, G2)], rows, sem)
        pltpu.sync_copy(ei1.at[pl.ds(base, G2)], nidx)
        cr.wait()
        pltpu.sync_copy(rows, nacc.at[nidx], add=True)
        return 0

    lax.fori_loop(0, 125, ch, 0)
    plsc.subcore_barrier()

    @pl.when(s < 15)
    def _():
        pltpu.sync_copy(nacc.at[pl.ds(s * 640, 640)],
                        npart.at[pl.ds(c * NR + s * 640, 640)])

    @pl.when(s == 15)
    def _():
        pltpu.sync_copy(nacc.at[pl.ds(9600, NR - 9600)],
                        npart.at[pl.ds(c * NR + 9600, NR - 9600)])


def _make_fs_kernel(EL, NR):
    EH = EL // 2
    mesh = plsc.VectorSubcoreMesh(core_axis_name="c", subcore_axis_name="s")
    body = functools.partial(_fs_body, EH, NR)
    return pl.kernel(
        body,
        out_type=jax.ShapeDtypeStruct((2 * NR, D), jnp.float32),
        scratch_types=[
            pltpu.VMEM((80,), jnp.int32),
            pltpu.VMEM((80, D), jnp.float32),
            pltpu.VMEM_SHARED((NR, D), jnp.float32),
            pltpu.SemaphoreType.DMA,
        ],
        mesh=mesh,
    )


def _xuv_body(wu, wv, x, xu, xv):
    xb = x[...]
    xu[...] = jnp.dot(xb, wu[...], preferred_element_type=jnp.float32)
    xv[...] = jnp.dot(xb, wv[...], preferred_element_type=jnp.float32)


def _ec_body(we, eattr, ec):
    ec[...] = jnp.dot(eattr[...], we[...], preferred_element_type=jnp.float32)


def _pr_body(wrel, wroot, o, p, r):
    blk = o[...]
    p[...] = jnp.dot(blk, wrel[...], preferred_element_type=jnp.float32)
    r[...] = jnp.dot(blk, wroot[...], preferred_element_type=jnp.float32)


def _m_body(q, r, seg, m, acc):
    j = pl.program_id(0)

    @pl.when(j == 0)
    def _():
        acc[...] = jnp.full((1, NG), -3e38, jnp.float32)

    xc = q[...] + r[...]
    oh = seg[...] == lax.broadcasted_iota(jnp.int32, (1, NG), 1)
    masked = jnp.where(oh, xc, -3e38)
    acc[...] = jnp.maximum(acc[...], jnp.max(masked, axis=0, keepdims=True))

    @pl.when(j == pl.num_programs(0) - 1)
    def _():
        m[...] = acc[...]


def _gx_body(m, wg, bg, an, q, r, seg, o, logit, accgx, accden):
    j = pl.program_id(0)

    @pl.when(j == 0)
    def _():
        accgx[...] = jnp.zeros((NG, NG), jnp.float32)
        accden[...] = jnp.zeros((NG, 1), jnp.float32)

    xc = q[...] + r[...]
    ohf = (seg[...] == lax.broadcasted_iota(jnp.int32, (1, NG), 1)
           ).astype(jnp.float32)
    mg = jnp.dot(ohf, m[...], preferred_element_type=jnp.float32)
    ex = jnp.exp(xc - mg)
    ohw = ohf * ex
    accgx[...] += lax.dot_general(ohw, o[...], (((0,), (0,)), ((), ())),
                                  preferred_element_type=jnp.float32)
    accden[...] += lax.dot_general(ohw, jnp.ones((M, 1), jnp.float32),
                                   (((0,), (0,)), ((), ())),
                                   preferred_element_type=jnp.float32)

    @pl.when(j == pl.num_programs(0) - 1)
    def _():
        gx = accgx[...] / (accden[...] + 1e-16)
        gout = jnp.tanh(jnp.dot(gx, wg[...],
                                preferred_element_type=jnp.float32) + bg[...])
        logit[...] = jnp.dot(gout, an[...], preferred_element_type=jnp.float32)


def _comb_body(lg, ab, seg, o1, o2, o3, o4, of):
    z = lg[...] + ab[...]
    z = z - jnp.max(z, axis=-1, keepdims=True)
    ez = jnp.exp(z)
    sc = ez / jnp.sum(ez, axis=-1, keepdims=True)
    ohf = (seg[...] == lax.broadcasted_iota(jnp.int32, (1, NG), 1)
           ).astype(jnp.float32)
    w = jnp.dot(ohf, sc, preferred_element_type=jnp.float32)
    of[...] = (w[:, 0:1] * o1[...] + w[:, 1:2] * o2[...]
               + w[:, 2:3] * o3[...] + w[:, 3:4] * o4[...])


def _xnew_body(x, n0, n1, xo):
    xo[...] = x[...] + n0[...] + n1[...]


def _blk(shape, imap):
    return pl.BlockSpec(shape, imap)


def _full(*_):
    return (0, 0)


def _rowj(j):
    return (j, 0)


def kernel(x, edge_index, edge_attr, edge_index_bond, edge_index_batch,
           W_u, W_v, W_e, W_rel, b_rel, W_root, a, W_gout, b_gout, a_bias):
    EL = edge_attr.shape[0]
    N = x.shape[0]
    src, dst = edge_index_bond[0], edge_index_bond[1]
    n_iter = a.shape[-1]

    C = 3200
    NB = EL // C
    EH = EL // 2
    EP2 = EL + G
    NR = 10016

    f32 = jnp.float32

    dst_s, src_s = lax.sort_key_val(dst.astype(jnp.int32), src.astype(jnp.int32))
    CB = C + 16
    ldst = (dst_s % C).astype(jnp.int32)
    ldst1 = (dst_s % EH).astype(jnp.int32)
    src_pad = jnp.concatenate([src_s, jnp.zeros((G,), jnp.int32)])
    ldh = [jnp.concatenate([ldst + h * CB,
                            jnp.full((G,), h * CB + C, jnp.int32)])
           for h in range(3)]
    ldst1_pad = jnp.concatenate([ldst1, jnp.full((G,), EH, jnp.int32)])
    offs = jnp.searchsorted(dst_s, jnp.arange(NB + 1, dtype=jnp.int32) * C)
    offs = jnp.concatenate([offs.astype(jnp.int32),
                            jnp.zeros((112 - NB - 1,), jnp.int32)])
    offs = jnp.tile(offs[:, None], (1, 16))
    qoffs = jnp.searchsorted(dst_s, jnp.arange(3, dtype=jnp.int32) * EH)
    qoffs = jnp.concatenate([qoffs.astype(jnp.int32),
                             jnp.zeros((13,), jnp.int32)])
    qoffs = jnp.tile(qoffs[:, None], (1, 16))

    ei0 = edge_index[0].astype(jnp.int32)
    ei1 = edge_index[1].astype(jnp.int32)
    seg2 = edge_index_batch.astype(jnp.int32).reshape(EL, 1)
    zn = jnp.zeros((NR, D), f32)

    mp = _make_mp_kernel(EL, C)
    qk = _make_q_kernel(EL, EH)
    eak = _make_ea_kernel(EL)
    fsk = _make_fs_kernel(EL, NR)

    NBLK = EL // M

    xu, xv = pl.pallas_call(
        _xuv_body,
        grid=(N // 1000,),
        in_specs=[_blk((D, D), _full), _blk((D, D), _full),
                  _blk((1000, D), _rowj)],
        out_specs=[_blk((1000, D), _rowj)] * 2,
        out_shape=[jax.ShapeDtypeStruct((N, D), f32)] * 2,
    )(W_u, W_v, x)

    ec = pl.pallas_call(
        _ec_body,
        grid=(NBLK,),
        in_specs=[_blk((16, D), _full), _blk((M, 16), _rowj)],
        out_specs=_blk((M, D), _rowj),
        out_shape=jax.ShapeDtypeStruct((EL, D), f32),
    )(W_e, edge_attr)

    ea = eak(xu, xv, ec, ei0, ei1)

    wrel = W_rel.astype(f32)
    wroot = W_root.astype(f32)
    bg = b_gout.reshape(1, D).astype(f32)
    ab = a_bias.reshape(1, n_iter).astype(f32)

    out = ea
    outs = []
    logits = []
    for n in range(n_iter):
        out = mp(out, ea, src_pad, ldh[0], ldh[1], ldh[2], offs)
        p2, r2 = pl.pallas_call(
            _pr_body,
            grid=(NBLK,),
            in_specs=[_blk((D, 1), _full), _blk((D, 1), _full),
                      _blk((M, D), _rowj)],
            out_specs=[_blk((M, 1), _rowj)] * 2,
            out_shape=[jax.ShapeDtypeStruct((EL, 1), f32)] * 2,
        )(wrel, wroot, out)
        q = qk(p2.reshape(EL), src_pad, ldst1_pad, qoffs)
        q2 = q.reshape(EL, 1)
        m = pl.pallas_call(
            _m_body,
            grid=(NBLK,),
            in_specs=[_blk((M, 1), _rowj), _blk((M, 1), _rowj),
                      _blk((M, 1), _rowj)],
            out_specs=_blk((1, NG), _full),
            out_shape=jax.ShapeDtypeStruct((1, NG), f32),
            scratch_shapes=[pltpu.VMEM((1, NG), f32)],
        )(q2, r2, seg2)
        logit = pl.pallas_call(
            _gx_body,
            grid=(NBLK,),
            in_specs=[_blk((NG, 1), _full), _blk((D, D), _full),
                      _blk((1, D), _full), _blk((D, 1), _full),
                      _blk((M, 1), _rowj), _blk((M, 1), _rowj),
                      _blk((M, 1), _rowj), _blk((M, D), _rowj)],
            out_specs=_blk((NG, 1), _full),
            out_shape=jax.ShapeDtypeStruct((NG, 1), f32),
            scratch_shapes=[pltpu.VMEM((NG, NG), f32),
                            pltpu.VMEM((NG, 1), f32)],
        )(m.reshape(NG, 1), W_gout.astype(f32), bg,
          a[0, :, n].reshape(D, 1).astype(f32), q2, r2, seg2, out)
        outs.append(out)
        logits.append(logit)

    lg = jnp.concatenate(logits, axis=1)

    out_final = pl.pallas_call(
        _comb_body,
        grid=(NBLK,),
        in_specs=[_blk((NG, n_iter), _full), _blk((1, n_iter), _full),
                  _blk((M, 1), _rowj), _blk((M, D), _rowj),
                  _blk((M, D), _rowj), _blk((M, D), _rowj),
                  _blk((M, D), _rowj)],
        out_specs=_blk((M, D), _rowj),
        out_shape=jax.ShapeDtypeStruct((EL, D), f32),
    )(lg, ab, seg2, *outs)

    npart = fsk(out_final, ei1, zn)

    x_new = pl.pallas_call(
        _xnew_body,
        grid=(N // 1000,),
        in_specs=[_blk((1000, D), _rowj), _blk((1000, D), _rowj),
                  _blk((1000, D), _rowj)],
        out_specs=_blk((1000, D), _rowj),
        out_shape=jax.ShapeDtypeStruct((N, D), f32),
    )(x, npart[:N], npart[NR:NR + N])
    return x_new
